# baseline (speedup 1.0000x reference)
import functools

import jax
import jax.numpy as jnp
from jax import lax
from jax.experimental import pallas as pl
from jax.experimental.pallas import tpu as pltpu
from jax.experimental.pallas import tpu_sc as plsc

B = 16384
L = 50
D = 64
NC = 2
NS = 16
NW = NC * NS
SPT = B // NW
NBUF = 4
NLANE = 16
NREG = D // NLANE


def _body(spans_hbm, we_hbm, out_hbm, idx_all, rows, out_v, *sems):
    wid = lax.axis_index("s") * NC + lax.axis_index("c")
    base = wid * SPT

    pltpu.sync_copy(spans_hbm.at[pl.ds(base, SPT), :], idx_all)

    def gather_start(s, b):
        pltpu.async_copy(we_hbm.at[idx_all.at[s]], rows.at[b], sems[b])

    def gather_wait(s, b):
        pltpu.make_async_copy(we_hbm.at[idx_all.at[s]], rows.at[b], sems[b]).wait()

    for b in range(NBUF):
        gather_start(b, b)

    @pl.loop(0, SPT, step=NBUF)
    def _ring(s0):
        for b in range(NBUF):
            s = s0 + b
            gather_wait(s, b)
            accs = tuple(jnp.zeros((NLANE,), jnp.float32) for _ in range(NREG))

            def _acc(j, accs, b=b):
                return tuple(
                    accs[d] + rows[b, j, pl.ds(d * NLANE, NLANE)]
                    for d in range(NREG)
                )

            accs = lax.fori_loop(0, L, _acc, accs, unroll=10)
            for d in range(NREG):
                out_v[s, pl.ds(d * NLANE, NLANE)] = accs[d]

            @pl.when(s + NBUF < SPT)
            def _():
                gather_start(s + NBUF, b)

    pltpu.sync_copy(out_v, out_hbm.at[pl.ds(base, SPT), :])


@jax.jit
def kernel(spans, we):
    spans2d = spans.astype(jnp.int32)
    mesh = plsc.VectorSubcoreMesh(
        core_axis_name="c", subcore_axis_name="s", num_cores=NC, num_subcores=NS
    )
    f = pl.kernel(
        _body,
        out_type=jax.ShapeDtypeStruct((B, D), jnp.float32),
        mesh=mesh,
        scratch_types=[
            pltpu.VMEM((SPT, L), jnp.int32),
            pltpu.VMEM((NBUF, L, D), jnp.float32),
            pltpu.VMEM((SPT, D), jnp.float32),
        ] + [pltpu.SemaphoreType.DMA] * NBUF,
        compiler_params=pltpu.CompilerParams(use_tc_tiling_on_sc=False),
    )
    return f(spans2d, we)


# speedup vs baseline: 1.1255x; 1.1255x over previous
import functools

import jax
import jax.numpy as jnp
from jax import lax
from jax.experimental import pallas as pl
from jax.experimental.pallas import tpu as pltpu
from jax.experimental.pallas import tpu_sc as plsc

B = 16384
L = 50
D = 64
NC = 2
NS = 16
NW = NC * NS
SPT = B // NW
S = 8
NCHUNK = SPT // S
NBUF = 4
NLANE = 16
NREG = D // NLANE


def _body(spans_hbm, we_hbm, out_hbm, idx_all, rows, out_v, *sems):
    wid = lax.axis_index("s") * NC + lax.axis_index("c")
    base = wid * SPT

    pltpu.sync_copy(spans_hbm.at[pl.ds(base * L, SPT * L)], idx_all)

    def gather_start(c, b):
        pltpu.async_copy(
            we_hbm.at[idx_all.at[pl.ds(c * (S * L), S * L)]], rows.at[b], sems[b]
        )

    def gather_wait(c, b):
        pltpu.make_async_copy(
            we_hbm.at[idx_all.at[pl.ds(c * (S * L), S * L)]], rows.at[b], sems[b]
        ).wait()

    def pool_chunk(c, b):
        for s in range(S):
            accs = tuple(jnp.zeros((NLANE,), jnp.float32) for _ in range(NREG))

            def _acc(j, accs, s=s, b=b):
                return tuple(
                    accs[d] + rows[b, s * L + j, pl.ds(d * NLANE, NLANE)]
                    for d in range(NREG)
                )

            accs = lax.fori_loop(0, L, _acc, accs, unroll=10)
            for d in range(NREG):
                out_v[s, pl.ds(d * NLANE, NLANE)] = accs[d]
        pltpu.sync_copy(out_v, out_hbm.at[pl.ds(base + c * S, S), :])

    gather_start(0, 0)
    gather_start(1, 1)

    @pl.loop(0, NCHUNK - NBUF, step=NBUF)
    def _ring(c0):
        for b in range(NBUF):
            c = c0 + b
            gather_wait(c, b)
            gather_start(c + 2, (b + 2) % NBUF)
            pool_chunk(c, b)

    for b in range(NBUF):
        c = NCHUNK - NBUF + b
        gather_wait(c, b)
        if b < 2:
            gather_start(c + 2, (b + 2) % NBUF)
        pool_chunk(c, b)


@jax.jit
def kernel(spans, we):
    spans_flat = spans.reshape(-1).astype(jnp.int32)
    mesh = plsc.VectorSubcoreMesh(
        core_axis_name="c", subcore_axis_name="s", num_cores=NC, num_subcores=NS
    )
    f = pl.kernel(
        _body,
        out_type=jax.ShapeDtypeStruct((B, D), jnp.float32),
        mesh=mesh,
        scratch_types=[
            pltpu.VMEM((SPT * L,), jnp.int32),
            pltpu.VMEM((NBUF, S * L, D), jnp.float32),
            pltpu.VMEM((S, D), jnp.float32),
        ] + [pltpu.SemaphoreType.DMA] * NBUF,
        compiler_params=pltpu.CompilerParams(use_tc_tiling_on_sc=False),
    )
    return f(spans_flat, we)
